# Initial kernel scaffold; baseline (speedup 1.0000x reference)
import functools

import jax
import jax.numpy as jnp
from jax import lax
from jax.experimental import pallas as pl
from jax.experimental.pallas import tpu as pltpu
from jax.experimental.pallas import tpu_sc as plsc

N = 10000
C = 128
E = 320000

NC = 2
NS = 16
NW = NC * NS
R = 80
EPAD = NW * R * 128
NPAD = 10240
STRIPE = NPAD // NS
NBUF = 4

_MESH = plsc.VectorSubcoreMesh(core_axis_name="c", subcore_axis_name="s")


@functools.partial(
    pl.kernel,
    out_type=jax.ShapeDtypeStruct((NC, NPAD), jnp.float32),
    mesh=_MESH,
    scratch_types=[
        pltpu.VMEM((R, 128), jnp.int32),
        pltpu.VMEM((128,), jnp.float32),
        pltpu.VMEM_SHARED((NPAD,), jnp.float32),
    ],
)
def _deg_kernel(dst_hbm, z_hbm, out_hbm, dst_v, ones_v, deg_sh):
    c = lax.axis_index("c")
    s = lax.axis_index("s")
    w = s * NC + c
    pltpu.sync_copy(z_hbm, deg_sh.at[pl.ds(s * STRIPE, STRIPE)])

    def set_ones(i, carry):
        ones_v[pl.ds(i * 16, 16)] = jnp.ones((16,), jnp.float32)
        return carry

    lax.fori_loop(0, 128 // 16, set_ones, 0)
    pltpu.sync_copy(dst_hbm.at[w], dst_v)
    plsc.subcore_barrier()

    def body(g, carry):
        pltpu.sync_copy(ones_v, deg_sh.at[dst_v.at[g]], add=True)
        return carry

    lax.fori_loop(0, R, body, 0)
    plsc.subcore_barrier()
    pltpu.sync_copy(
        deg_sh.at[pl.ds(s * STRIPE, STRIPE)],
        out_hbm.at[c, pl.ds(s * STRIPE, STRIPE)],
    )


@functools.partial(
    pl.kernel,
    out_type=jax.ShapeDtypeStruct((NC, NPAD, C), jnp.float32),
    mesh=_MESH,
    scratch_types=[
        pltpu.VMEM((R, 128), jnp.int32),
        pltpu.VMEM((R, 128), jnp.int32),
        pltpu.VMEM((128, C), jnp.float32),
        pltpu.VMEM((128, C), jnp.float32),
        pltpu.VMEM((128, C), jnp.float32),
        pltpu.VMEM((128, C), jnp.float32),
        pltpu.VMEM_SHARED((NPAD, C), jnp.float32),
        pltpu.SemaphoreType.DMA,
        pltpu.SemaphoreType.DMA,
        pltpu.SemaphoreType.DMA,
        pltpu.SemaphoreType.DMA,
    ],
)
def _agg_kernel(y_hbm, src_hbm, dst_hbm, zz_hbm, out_hbm,
                src_v, dst_v, b0, b1, b2, b3, acc_sh, s0, s1, s2, s3):
    c = lax.axis_index("c")
    s = lax.axis_index("s")
    w = s * NC + c
    bufs = [b0, b1, b2, b3]
    sems = [s0, s1, s2, s3]

    pltpu.sync_copy(zz_hbm, acc_sh.at[pl.ds(s * STRIPE, STRIPE)])
    pltpu.sync_copy(src_hbm.at[w], src_v)
    pltpu.sync_copy(dst_hbm.at[w], dst_v)
    plsc.subcore_barrier()

    for b in range(NBUF - 1):
        pltpu.async_copy(y_hbm.at[src_v.at[b]], bufs[b], sems[b])

    def outer(i, carry):
        base = i * NBUF
        for b in range(NBUF):
            g = base + b
            pltpu.make_async_copy(y_hbm.at[src_v.at[g]], bufs[b], sems[b]).wait()
            pltpu.sync_copy(bufs[b], acc_sh.at[dst_v.at[g]], add=True)
            nb = (b + NBUF - 1) % NBUF

            @pl.when(g + NBUF - 1 < R)
            def _():
                pltpu.async_copy(
                    y_hbm.at[src_v.at[g + NBUF - 1]], bufs[nb], sems[nb])
        return carry

    lax.fori_loop(0, R // NBUF, outer, 0)
    plsc.subcore_barrier()
    pltpu.sync_copy(
        acc_sh.at[pl.ds(s * STRIPE, STRIPE)],
        out_hbm.at[c, pl.ds(s * STRIPE, STRIPE)],
    )


BLK = 2000


def _prescale_body(feats_ref, w_ref, degs_ref, y_ref):
    deg = degs_ref[0, :] + degs_ref[1, :] + 1.0
    dinv = lax.rsqrt(deg)
    x = jnp.dot(feats_ref[...], w_ref[...], preferred_element_type=jnp.float32)
    y_ref[...] = x * dinv[:, None]


def _final_body(accs_ref, y_ref, degs_ref, b_ref, out_ref):
    acc = accs_ref[0] + accs_ref[1]
    deg = degs_ref[0, :] + degs_ref[1, :] + 1.0
    dinv = lax.rsqrt(deg)
    out = dinv[:, None] * (acc + y_ref[...]) + b_ref[0, :][None, :]
    out_ref[...] = jnp.maximum(out, 0.0)


def kernel(feats, edges, W, b):
    src = edges[0].astype(jnp.int32)
    dst = edges[1].astype(jnp.int32)
    npad_e = EPAD - E
    src_t = jnp.concatenate(
        [src, jnp.zeros((npad_e,), jnp.int32)]).reshape(NW, R, 128)
    dst_t = jnp.concatenate(
        [dst, jnp.full((npad_e,), N, jnp.int32)]).reshape(NW, R, 128)
    z1 = jnp.zeros((STRIPE,), jnp.float32)
    zz = jnp.zeros((STRIPE, C), jnp.float32)

    degs = _deg_kernel(dst_t, z1)

    y = pl.pallas_call(
        _prescale_body,
        grid=(N // BLK,),
        in_specs=[
            pl.BlockSpec((BLK, C), lambda i: (i, 0)),
            pl.BlockSpec((C, C), lambda i: (0, 0)),
            pl.BlockSpec((2, BLK), lambda i: (0, i)),
        ],
        out_specs=pl.BlockSpec((BLK, C), lambda i: (i, 0)),
        out_shape=jax.ShapeDtypeStruct((N, C), jnp.float32),
    )(feats, W, degs)

    accs = _agg_kernel(y, src_t, dst_t, zz)

    out = pl.pallas_call(
        _final_body,
        grid=(N // BLK,),
        in_specs=[
            pl.BlockSpec((2, BLK, C), lambda i: (0, i, 0)),
            pl.BlockSpec((BLK, C), lambda i: (i, 0)),
            pl.BlockSpec((2, BLK), lambda i: (0, i)),
            pl.BlockSpec((1, C), lambda i: (0, 0)),
        ],
        out_specs=pl.BlockSpec((BLK, C), lambda i: (i, 0)),
        out_shape=jax.ShapeDtypeStruct((N, C), jnp.float32),
    )(accs, y, degs, b.reshape(1, C))
    return out


# speedup vs baseline: 16.8376x; 16.8376x over previous
import functools

import jax
import jax.numpy as jnp
from jax import lax
from jax.experimental import pallas as pl
from jax.experimental.pallas import tpu as pltpu
from jax.experimental.pallas import tpu_sc as plsc

N = 10000
C = 128
E = 320000

NC = 2
NS = 16
NW = NC * NS
R = 80
EPAD = NW * R * 128
NPAD = 10240
STRIPE = NPAD // NS
NBUF = 4

_MESH = plsc.VectorSubcoreMesh(core_axis_name="c", subcore_axis_name="s")


@functools.partial(
    pl.kernel,
    out_type=jax.ShapeDtypeStruct((NC, NPAD), jnp.float32),
    mesh=_MESH,
    scratch_types=[
        pltpu.VMEM((R, 128), jnp.int32),
        pltpu.VMEM((128,), jnp.float32),
        pltpu.VMEM_SHARED((NPAD,), jnp.float32),
    ],
)
def _deg_kernel(dst_hbm, z_hbm, out_hbm, dst_v, ones_v, deg_sh):
    c = lax.axis_index("c")
    s = lax.axis_index("s")
    w = s * NC + c
    pltpu.sync_copy(z_hbm, deg_sh.at[pl.ds(s * STRIPE, STRIPE)])

    def set_ones(i, carry):
        ones_v[pl.ds(i * 16, 16)] = jnp.ones((16,), jnp.float32)
        return carry

    lax.fori_loop(0, 128 // 16, set_ones, 0)
    pltpu.sync_copy(dst_hbm.at[w], dst_v)
    plsc.subcore_barrier()

    def body(g, carry):
        pltpu.sync_copy(ones_v, deg_sh.at[dst_v.at[g]], add=True)
        return carry

    lax.fori_loop(0, R, body, 0)
    plsc.subcore_barrier()
    pltpu.sync_copy(
        deg_sh.at[pl.ds(s * STRIPE, STRIPE)],
        out_hbm.at[c, pl.ds(s * STRIPE, STRIPE)],
    )


CH = C // NC
R2 = EPAD // 128 // NS


@functools.partial(
    pl.kernel,
    out_type=jax.ShapeDtypeStruct((NC, NPAD, CH), jnp.float32),
    mesh=_MESH,
    scratch_types=[
        pltpu.VMEM((R2, 128), jnp.int32),
        pltpu.VMEM((R2, 128), jnp.int32),
        pltpu.VMEM((128, CH), jnp.float32),
        pltpu.VMEM((128, CH), jnp.float32),
        pltpu.VMEM((128, CH), jnp.float32),
        pltpu.VMEM((128, CH), jnp.float32),
        pltpu.VMEM_SHARED((NPAD, CH), jnp.float32),
        pltpu.SemaphoreType.DMA,
        pltpu.SemaphoreType.DMA,
        pltpu.SemaphoreType.DMA,
        pltpu.SemaphoreType.DMA,
    ],
    compiler_params=pltpu.CompilerParams(use_tc_tiling_on_sc=False),
)
def _agg_kernel(y2_hbm, idx2_hbm, dst_hbm, zz_hbm, out_hbm,
                idx_v, dst_v, b0, b1, b2, b3, acc_sh, s0, s1, s2, s3):
    c = lax.axis_index("c")
    s = lax.axis_index("s")
    bufs = [b0, b1, b2, b3]
    sems = [s0, s1, s2, s3]

    pltpu.sync_copy(zz_hbm, acc_sh.at[pl.ds(s * STRIPE, STRIPE)])
    pltpu.sync_copy(idx2_hbm.at[c, s], idx_v)
    pltpu.sync_copy(dst_hbm.at[s], dst_v)
    plsc.subcore_barrier()

    for b in range(NBUF - 1):
        pltpu.async_copy(y2_hbm.at[idx_v.at[b]], bufs[b], sems[b])

    def outer(i, carry):
        base = i * NBUF
        for b in range(NBUF):
            g = base + b
            pltpu.make_async_copy(y2_hbm.at[idx_v.at[g]], bufs[b], sems[b]).wait()
            pltpu.sync_copy(bufs[b], acc_sh.at[dst_v.at[g]], add=True)
            nb = (b + NBUF - 1) % NBUF

            @pl.when(g + NBUF - 1 < R2)
            def _():
                pltpu.async_copy(
                    y2_hbm.at[idx_v.at[g + NBUF - 1]], bufs[nb], sems[nb])
        return carry

    lax.fori_loop(0, R2 // NBUF, outer, 0)
    plsc.subcore_barrier()
    pltpu.sync_copy(
        acc_sh.at[pl.ds(s * STRIPE, STRIPE)],
        out_hbm.at[c, pl.ds(s * STRIPE, STRIPE)],
    )


BLK = 2000


def _prescale_body(feats_ref, w_ref, degs_ref, y_ref):
    deg = degs_ref[:, 0:1] + degs_ref[:, 1:2] + 1.0
    dinv = lax.rsqrt(deg)
    x = jnp.dot(feats_ref[...], w_ref[...], preferred_element_type=jnp.float32)
    y_ref[...] = x * dinv


def _final_body(accs_ref, y_ref, degs_ref, b_ref, out_ref):
    acc = jnp.concatenate([accs_ref[0], accs_ref[1]], axis=-1)
    deg = degs_ref[:, 0:1] + degs_ref[:, 1:2] + 1.0
    dinv = lax.rsqrt(deg)
    out = dinv * (acc + y_ref[...]) + b_ref[0, :][None, :]
    out_ref[...] = jnp.maximum(out, 0.0)


def kernel(feats, edges, W, b):
    src = edges[0].astype(jnp.int32)
    dst = edges[1].astype(jnp.int32)
    npad_e = EPAD - E
    src_p = jnp.concatenate([src, jnp.zeros((npad_e,), jnp.int32)])
    dst_p = jnp.concatenate([dst, jnp.full((npad_e,), N, jnp.int32)])
    dst_t = dst_p.reshape(NW, R, 128)
    dst_t2 = dst_p.reshape(NS, R2, 128)
    src2 = (src_p * 2).reshape(NS, R2, 128)
    idx2 = jnp.stack([src2, src2 + 1], axis=0)
    z1 = jnp.zeros((STRIPE,), jnp.float32)
    zz = jnp.zeros((STRIPE, CH), jnp.float32)

    degs = _deg_kernel(dst_t, z1)
    degs_t = jnp.transpose(degs)

    y = pl.pallas_call(
        _prescale_body,
        grid=(N // BLK,),
        in_specs=[
            pl.BlockSpec((BLK, C), lambda i: (i, 0)),
            pl.BlockSpec((C, C), lambda i: (0, 0)),
            pl.BlockSpec((BLK, 2), lambda i: (i, 0)),
        ],
        out_specs=pl.BlockSpec((BLK, C), lambda i: (i, 0)),
        out_shape=jax.ShapeDtypeStruct((N, C), jnp.float32),
    )(feats, W, degs_t)

    y2 = y.reshape(2 * N, CH)
    accs = _agg_kernel(y2, idx2, dst_t2, zz)

    out = pl.pallas_call(
        _final_body,
        grid=(N // BLK,),
        in_specs=[
            pl.BlockSpec((2, BLK, CH), lambda i: (0, i, 0)),
            pl.BlockSpec((BLK, C), lambda i: (i, 0)),
            pl.BlockSpec((BLK, 2), lambda i: (i, 0)),
            pl.BlockSpec((1, C), lambda i: (0, 0)),
        ],
        out_specs=pl.BlockSpec((BLK, C), lambda i: (i, 0)),
        out_shape=jax.ShapeDtypeStruct((N, C), jnp.float32),
    )(accs, y, degs_t, b.reshape(1, C))
    return out


# speedup vs baseline: 16.8464x; 1.0005x over previous
import functools

import jax
import jax.numpy as jnp
from jax import lax
from jax.experimental import pallas as pl
from jax.experimental.pallas import tpu as pltpu
from jax.experimental.pallas import tpu_sc as plsc

N = 10000
C = 128
E = 320000

NC = 2
NS = 16
NW = NC * NS
R = 80
EPAD = NW * R * 128
NPAD = 10240
STRIPE = NPAD // NS
NBUF = 4

_MESH = plsc.VectorSubcoreMesh(core_axis_name="c", subcore_axis_name="s")


@functools.partial(
    pl.kernel,
    out_type=jax.ShapeDtypeStruct((NC, NPAD), jnp.float32),
    mesh=_MESH,
    scratch_types=[
        pltpu.VMEM((R, 128), jnp.int32),
        pltpu.VMEM((128,), jnp.float32),
        pltpu.VMEM_SHARED((NPAD,), jnp.float32),
    ],
)
def _deg_kernel(dst_hbm, z_hbm, out_hbm, dst_v, ones_v, deg_sh):
    c = lax.axis_index("c")
    s = lax.axis_index("s")
    w = s * NC + c
    pltpu.sync_copy(z_hbm, deg_sh.at[pl.ds(s * STRIPE, STRIPE)])

    def set_ones(i, carry):
        ones_v[pl.ds(i * 16, 16)] = jnp.ones((16,), jnp.float32)
        return carry

    lax.fori_loop(0, 128 // 16, set_ones, 0)
    pltpu.sync_copy(dst_hbm.at[w], dst_v)
    plsc.subcore_barrier()

    def body(g, carry):
        pltpu.sync_copy(ones_v, deg_sh.at[dst_v.at[g]], add=True)
        return carry

    lax.fori_loop(0, R, body, 0)
    plsc.subcore_barrier()
    pltpu.sync_copy(
        deg_sh.at[pl.ds(s * STRIPE, STRIPE)],
        out_hbm.at[c, pl.ds(s * STRIPE, STRIPE)],
    )


CH = C // NC
R2 = EPAD // 128 // NS


@functools.partial(
    pl.kernel,
    out_type=jax.ShapeDtypeStruct((NC, NPAD, CH), jnp.float32),
    mesh=_MESH,
    scratch_types=[
        pltpu.VMEM((R2, 128), jnp.int32),
        pltpu.VMEM((R2, 128), jnp.int32),
    ] + [pltpu.VMEM((128, CH), jnp.float32)] * 5
      + [pltpu.VMEM_SHARED((NPAD, CH), jnp.float32)]
      + [pltpu.SemaphoreType.DMA] * 10,
    compiler_params=pltpu.CompilerParams(use_tc_tiling_on_sc=False),
)
def _agg_kernel(y2_hbm, idx2_hbm, dst_hbm, zz_hbm, out_hbm,
                idx_v, dst_v, *rest):
    c = lax.axis_index("c")
    s = lax.axis_index("s")
    bufs = list(rest[0:5])
    acc_sh = rest[5]
    gsems = list(rest[6:11])
    ssems = list(rest[11:16])
    NB = 5
    AHEAD = 3

    pltpu.sync_copy(zz_hbm, acc_sh.at[pl.ds(s * STRIPE, STRIPE)])
    pltpu.sync_copy(idx2_hbm.at[c, s], idx_v)
    pltpu.sync_copy(dst_hbm.at[s], dst_v)
    plsc.subcore_barrier()

    def gather(g, slot):
        pltpu.async_copy(y2_hbm.at[idx_v.at[g]], bufs[slot], gsems[slot])

    def gather_wait(g, slot):
        pltpu.make_async_copy(
            y2_hbm.at[idx_v.at[g]], bufs[slot], gsems[slot]).wait()

    def scatter(g, slot):
        pltpu.async_copy(
            bufs[slot], acc_sh.at[dst_v.at[g]], ssems[slot], add=True)

    def scatter_wait(g, slot):
        pltpu.make_async_copy(
            bufs[slot], acc_sh.at[dst_v.at[g]], ssems[slot]).wait()

    for b in range(AHEAD):
        gather(b, b)

    def outer(i, carry):
        base = i * NB
        for b in range(NB):
            g = base + b
            gather_wait(g, b)
            scatter(g, b)
            @pl.when(g + AHEAD < R2)
            def _():
                @pl.when(g >= NB - AHEAD)
                def _():
                    scatter_wait(g - (NB - AHEAD), (b + AHEAD) % NB)

                gather(g + AHEAD, (b + AHEAD) % NB)
        return carry

    lax.fori_loop(0, R2 // NB, outer, 0)
    for b in range(NB):
        scatter_wait(R2 - NB + b, b)
    plsc.subcore_barrier()
    pltpu.sync_copy(
        acc_sh.at[pl.ds(s * STRIPE, STRIPE)],
        out_hbm.at[c, pl.ds(s * STRIPE, STRIPE)],
    )


BLK = 2000


def _prescale_body(feats_ref, w_ref, degs_ref, y_ref):
    deg = degs_ref[:, 0:1] + degs_ref[:, 1:2] + 1.0
    dinv = lax.rsqrt(deg)
    x = jnp.dot(feats_ref[...], w_ref[...], preferred_element_type=jnp.float32)
    y_ref[...] = x * dinv


def _final_body(accs_ref, y_ref, degs_ref, b_ref, out_ref):
    acc = jnp.concatenate([accs_ref[0], accs_ref[1]], axis=-1)
    deg = degs_ref[:, 0:1] + degs_ref[:, 1:2] + 1.0
    dinv = lax.rsqrt(deg)
    out = dinv * (acc + y_ref[...]) + b_ref[0, :][None, :]
    out_ref[...] = jnp.maximum(out, 0.0)


def kernel(feats, edges, W, b):
    src = edges[0].astype(jnp.int32)
    dst = edges[1].astype(jnp.int32)
    npad_e = EPAD - E
    src_p = jnp.concatenate([src, jnp.zeros((npad_e,), jnp.int32)])
    dst_p = jnp.concatenate([dst, jnp.full((npad_e,), N, jnp.int32)])
    dst_t = dst_p.reshape(NW, R, 128)
    dst_t2 = dst_p.reshape(NS, R2, 128)
    src2 = (src_p * 2).reshape(NS, R2, 128)
    idx2 = jnp.stack([src2, src2 + 1], axis=0)
    z1 = jnp.zeros((STRIPE,), jnp.float32)
    zz = jnp.zeros((STRIPE, CH), jnp.float32)

    degs = _deg_kernel(dst_t, z1)
    degs_t = jnp.transpose(degs)

    y = pl.pallas_call(
        _prescale_body,
        grid=(N // BLK,),
        in_specs=[
            pl.BlockSpec((BLK, C), lambda i: (i, 0)),
            pl.BlockSpec((C, C), lambda i: (0, 0)),
            pl.BlockSpec((BLK, 2), lambda i: (i, 0)),
        ],
        out_specs=pl.BlockSpec((BLK, C), lambda i: (i, 0)),
        out_shape=jax.ShapeDtypeStruct((N, C), jnp.float32),
    )(feats, W, degs_t)

    y2 = y.reshape(2 * N, CH)
    accs = _agg_kernel(y2, idx2, dst_t2, zz)

    out = pl.pallas_call(
        _final_body,
        grid=(N // BLK,),
        in_specs=[
            pl.BlockSpec((2, BLK, CH), lambda i: (0, i, 0)),
            pl.BlockSpec((BLK, C), lambda i: (i, 0)),
            pl.BlockSpec((BLK, 2), lambda i: (i, 0)),
            pl.BlockSpec((1, C), lambda i: (0, 0)),
        ],
        out_specs=pl.BlockSpec((BLK, C), lambda i: (i, 0)),
        out_shape=jax.ShapeDtypeStruct((N, C), jnp.float32),
    )(accs, y, degs_t, b.reshape(1, C))
    return out
